# Initial kernel scaffold; baseline (speedup 1.0000x reference)
import jax
import jax.numpy as jnp
from jax.experimental import pallas as pl


def kernel(input_ids, token_table, pos_table):
    raise NotImplementedError("write your pallas kernel here")


# speedup vs baseline: 2.4185x; 2.4185x over previous
import functools

import jax
import jax.numpy as jnp
from jax import lax
from jax.experimental import pallas as pl
from jax.experimental.pallas import tpu as pltpu
from jax.experimental.pallas import tpu_sc as plsc

MAXLEN = 200
VOCAB = 100000
D = 128
BATCH = 1024

NC = 2
NS = 16
NW = NC * NS

N = BATCH * MAXLEN
RW = N // NW
CHUNK = 128
NCHUNK = RW // CHUNK
LANES = 16
SUBV = D // LANES


def _body(ids_hbm, tok_hbm, pos_hbm, out_hbm, idx_v, pos_v, buf_v, gsem):
    wid = lax.axis_index("s") * NC + lax.axis_index("c")
    base = wid * RW

    pltpu.sync_copy(ids_hbm.at[wid], idx_v)
    pltpu.sync_copy(pos_hbm, pos_v)

    @pl.loop(0, NCHUNK)
    def _chunk(c):
        pltpu.async_copy(tok_hbm.at[idx_v.at[c]], buf_v, gsem).wait()

        @pl.loop(0, CHUNK)
        def _row(i):
            p = lax.rem(c * CHUNK + i, MAXLEN)
            for k in range(SUBV):
                sl = pl.ds(k * LANES, LANES)
                plsc.addupdate(buf_v.at[i, sl], pos_v[p, sl])

        pltpu.sync_copy(buf_v, out_hbm.at[pl.ds(base + c * CHUNK, CHUNK)])


def kernel(input_ids, token_table, pos_table):
    ids = input_ids.astype(jnp.int32).reshape(NW, NCHUNK, CHUNK)
    mesh = plsc.VectorSubcoreMesh(
        core_axis_name="c", subcore_axis_name="s", num_cores=NC, num_subcores=NS
    )
    run = pl.kernel(
        _body,
        out_type=jax.ShapeDtypeStruct((N, D), jnp.float32),
        mesh=mesh,
        scratch_types=[
            pltpu.VMEM((N // CHUNK // NW, CHUNK), jnp.int32),
            pltpu.VMEM((MAXLEN, D), jnp.float32),
            pltpu.VMEM((CHUNK, D), jnp.float32),
            pltpu.SemaphoreType.DMA,
        ],
    )
    out = run(ids, token_table, pos_table)
    return out.reshape(BATCH, MAXLEN, D)


# speedup vs baseline: 3.1583x; 1.3059x over previous
import functools

import jax
import jax.numpy as jnp
from jax import lax
from jax.experimental import pallas as pl
from jax.experimental.pallas import tpu as pltpu
from jax.experimental.pallas import tpu_sc as plsc

MAXLEN = 200
VOCAB = 100000
D = 128
BATCH = 1024

NC = 2
NS = 16
NW = NC * NS

N = BATCH * MAXLEN
RW = N // NW
CHUNK = 128
NCHUNK = RW // CHUNK
LANES = 16
SUBV = D // LANES


def _body(ids_hbm, tok_hbm, pos_hbm, out_hbm, idx_v, pos_v, buf0_v, buf1_v, sem0, sem1):
    wid = lax.axis_index("s") * NC + lax.axis_index("c")
    base = wid * RW
    bufs = (buf0_v, buf1_v)
    sems = (sem0, sem1)

    pltpu.sync_copy(ids_hbm.at[wid], idx_v)
    pltpu.sync_copy(pos_hbm, pos_v)

    pltpu.async_copy(tok_hbm.at[idx_v.at[0]], buf0_v, sem0)

    @pl.loop(0, NCHUNK, step=2)
    def _group(g):
        for b in range(2):
            c = g + b
            buf, sem = bufs[b], sems[b]

            @pl.when(c + 1 < NCHUNK)
            def _():
                pltpu.async_copy(
                    tok_hbm.at[idx_v.at[c + 1]], bufs[1 - b], sems[1 - b]
                )

            pltpu.make_async_copy(tok_hbm.at[idx_v.at[c]], buf, sem).wait()

            @pl.loop(0, CHUNK)
            def _row(i):
                p = lax.rem(c * CHUNK + i, MAXLEN)
                for k in range(SUBV):
                    sl = pl.ds(k * LANES, LANES)
                    plsc.addupdate(buf.at[i, sl], pos_v[p, sl])

            pltpu.sync_copy(buf, out_hbm.at[pl.ds(base + c * CHUNK, CHUNK)])


def kernel(input_ids, token_table, pos_table):
    ids = input_ids.astype(jnp.int32).reshape(NW, NCHUNK, CHUNK)
    mesh = plsc.VectorSubcoreMesh(
        core_axis_name="c", subcore_axis_name="s", num_cores=NC, num_subcores=NS
    )
    run = pl.kernel(
        _body,
        out_type=jax.ShapeDtypeStruct((N, D), jnp.float32),
        mesh=mesh,
        scratch_types=[
            pltpu.VMEM((N // CHUNK // NW, CHUNK), jnp.int32),
            pltpu.VMEM((MAXLEN, D), jnp.float32),
            pltpu.VMEM((CHUNK, D), jnp.float32),
            pltpu.VMEM((CHUNK, D), jnp.float32),
            pltpu.SemaphoreType.DMA,
            pltpu.SemaphoreType.DMA,
        ],
    )
    out = run(ids, token_table, pos_table)
    return out.reshape(BATCH, MAXLEN, D)


# speedup vs baseline: 3.1908x; 1.0103x over previous
import functools

import jax
import jax.numpy as jnp
from jax import lax
from jax.experimental import pallas as pl
from jax.experimental.pallas import tpu as pltpu
from jax.experimental.pallas import tpu_sc as plsc

MAXLEN = 200
VOCAB = 100000
D = 128
BATCH = 1024

NC = 2
NS = 16
NW = NC * NS

N = BATCH * MAXLEN
RW = N // NW
CHUNK = 128
NCHUNK = RW // CHUNK
LANES = 16
SUBV = D // LANES


def _body(ids_hbm, tok_hbm, pos_hbm, out_hbm, idx_v, pos_v, buf0_v, buf1_v, sem0, sem1):
    wid = lax.axis_index("s") * NC + lax.axis_index("c")
    base = wid * RW
    bufs = (buf0_v, buf1_v)
    sems = (sem0, sem1)

    pltpu.sync_copy(ids_hbm.at[wid], idx_v)
    pltpu.sync_copy(pos_hbm, pos_v)

    pltpu.async_copy(tok_hbm.at[idx_v.at[0]], buf0_v, sem0)

    @pl.loop(0, NCHUNK, step=2)
    def _group(g):
        for b in range(2):
            c = g + b
            buf, sem = bufs[b], sems[b]

            @pl.when(c + 1 < NCHUNK)
            def _():
                pltpu.async_copy(
                    tok_hbm.at[idx_v.at[c + 1]], bufs[1 - b], sems[1 - b]
                )

            pltpu.make_async_copy(tok_hbm.at[idx_v.at[c]], buf, sem).wait()

            @pl.loop(0, CHUNK, unroll=8)
            def _row(i):
                p = lax.rem(c * CHUNK + i, MAXLEN)
                for k in range(SUBV):
                    sl = pl.ds(k * LANES, LANES)
                    plsc.addupdate(buf.at[i, sl], pos_v[p, sl])

            pltpu.sync_copy(buf, out_hbm.at[pl.ds(base + c * CHUNK, CHUNK)])


def kernel(input_ids, token_table, pos_table):
    ids = input_ids.astype(jnp.int32).reshape(NW, NCHUNK, CHUNK)
    mesh = plsc.VectorSubcoreMesh(
        core_axis_name="c", subcore_axis_name="s", num_cores=NC, num_subcores=NS
    )
    run = pl.kernel(
        _body,
        out_type=jax.ShapeDtypeStruct((N, D), jnp.float32),
        mesh=mesh,
        scratch_types=[
            pltpu.VMEM((N // CHUNK // NW, CHUNK), jnp.int32),
            pltpu.VMEM((MAXLEN, D), jnp.float32),
            pltpu.VMEM((CHUNK, D), jnp.float32),
            pltpu.VMEM((CHUNK, D), jnp.float32),
            pltpu.SemaphoreType.DMA,
            pltpu.SemaphoreType.DMA,
        ],
    )
    out = run(ids, token_table, pos_table)
    return out.reshape(BATCH, MAXLEN, D)
